# Initial kernel scaffold; baseline (speedup 1.0000x reference)
import jax
import jax.numpy as jnp
from jax.experimental import pallas as pl


def kernel(x, edge_index, W1l, b1l, W1r, W2l, b2l, W2r, Wc, bc, theta, Wcl, bcl):
    raise NotImplementedError("write your pallas kernel here")


# speedup vs baseline: 3.5964x; 3.5964x over previous
import functools

import jax
import jax.numpy as jnp
from jax import lax
from jax.experimental import pallas as pl
from jax.experimental.pallas import tpu as pltpu
from jax.experimental.pallas import tpu_sc as plsc

N = 10000
E = 320000
D = 128
DP = 144
NC, NS = 2, 16
NW = NC * NS
CB = 128
EP = 2560 * CB
CPW = (EP // CB) // NW
NPAD = 10240
RPS = NPAD // NS
BN = 1000


def _sc_agg_body(d, vals_hbm, src_hbm, dst_hbm, out0, out1,
                 sidx_v, didx_v, rows_v, agg_sh):
    c = lax.axis_index("c")
    s = lax.axis_index("s")
    w = c * NS + s

    def zrow(r, _):
        for j in range(d // 16):
            rows_v[r, pl.ds(j * 16, 16)] = jnp.zeros((16,), jnp.float32)
        return _
    lax.fori_loop(0, CB, zrow, 0)
    base = s * RPS
    for i in range(RPS // CB):
        pltpu.sync_copy(rows_v, agg_sh.at[pl.ds(base + i * CB, CB)])
    plsc.subcore_barrier()

    pltpu.sync_copy(src_hbm.at[pl.ds(w * CPW, CPW)], sidx_v)
    pltpu.sync_copy(dst_hbm.at[pl.ds(w * CPW, CPW)], didx_v)

    def chunk(j, _):
        pltpu.sync_copy(vals_hbm.at[sidx_v.at[j]], rows_v)
        pltpu.sync_copy(rows_v, agg_sh.at[didx_v.at[j]], add=True)
        return _
    lax.fori_loop(0, CPW, chunk, 0)
    plsc.subcore_barrier()

    @pl.when(c == 0)
    def _():
        for i in range(RPS // 80):
            @pl.when(base + i * 80 < N)
            def _():
                pltpu.sync_copy(agg_sh.at[pl.ds(base + i * 80, 80)],
                                out0.at[pl.ds(base + i * 80, 80)])

    @pl.when(c == 1)
    def _():
        for i in range(RPS // 80):
            @pl.when(base + i * 80 < N)
            def _():
                pltpu.sync_copy(agg_sh.at[pl.ds(base + i * 80, 80)],
                                out1.at[pl.ds(base + i * 80, 80)])


@functools.lru_cache(maxsize=None)
def _make_sc_agg(d):
    mesh = plsc.VectorSubcoreMesh(core_axis_name="c", subcore_axis_name="s",
                                  num_cores=NC, num_subcores=NS)
    return pl.kernel(
        functools.partial(_sc_agg_body, d),
        out_type=[jax.ShapeDtypeStruct((N, d), jnp.float32),
                  jax.ShapeDtypeStruct((N, d), jnp.float32)],
        mesh=mesh,
        compiler_params=pltpu.CompilerParams(use_tc_tiling_on_sc=False),
        scratch_types=[
            pltpu.VMEM((CPW, CB), jnp.int32),
            pltpu.VMEM((CPW, CB), jnp.int32),
            pltpu.VMEM((CB, d), jnp.float32),
            pltpu.VMEM_SHARED((NPAD, d), jnp.float32),
        ],
    )


def _tc1_body(a0_ref, a1_ref, x_ref, wl_ref, wr_ref, b_ref, h_ref, inv_ref):
    agg = a0_ref[...] + a1_ref[...]
    inv = 1.0 / jnp.maximum(agg[:, 128:129], 1.0)
    mean = agg[:, :128] * inv
    h = jnp.dot(mean, wl_ref[...], preferred_element_type=jnp.float32)
    h = h + jnp.dot(x_ref[...], wr_ref[...], preferred_element_type=jnp.float32)
    h = h + b_ref[...]
    h_ref[...] = jnp.maximum(h, 0.0)
    inv_ref[...] = inv


_tc1 = pl.pallas_call(
    _tc1_body,
    grid=(N // BN,),
    in_specs=[
        pl.BlockSpec((BN, DP), lambda i: (i, 0)),
        pl.BlockSpec((BN, DP), lambda i: (i, 0)),
        pl.BlockSpec((BN, D), lambda i: (i, 0)),
        pl.BlockSpec((D, D), lambda i: (0, 0)),
        pl.BlockSpec((D, D), lambda i: (0, 0)),
        pl.BlockSpec((1, D), lambda i: (0, 0)),
    ],
    out_specs=[pl.BlockSpec((BN, D), lambda i: (i, 0)),
               pl.BlockSpec((BN, 1), lambda i: (i, 0))],
    out_shape=[jax.ShapeDtypeStruct((N, D), jnp.float32),
               jax.ShapeDtypeStruct((N, 1), jnp.float32)],
)


def _tc2_body(a0_ref, a1_ref, h_ref, inv_ref, wl_ref, wr_ref, b_ref, h2_ref):
    mean = (a0_ref[...] + a1_ref[...]) * inv_ref[...]
    h2 = jnp.dot(mean, wl_ref[...], preferred_element_type=jnp.float32)
    h2 = h2 + jnp.dot(h_ref[...], wr_ref[...], preferred_element_type=jnp.float32)
    h2_ref[...] = jnp.maximum(h2 + b_ref[...], 0.0)


_tc2 = pl.pallas_call(
    _tc2_body,
    grid=(N // BN,),
    in_specs=[
        pl.BlockSpec((BN, D), lambda i: (i, 0)),
        pl.BlockSpec((BN, D), lambda i: (i, 0)),
        pl.BlockSpec((BN, D), lambda i: (i, 0)),
        pl.BlockSpec((BN, 1), lambda i: (i, 0)),
        pl.BlockSpec((D, D), lambda i: (0, 0)),
        pl.BlockSpec((D, D), lambda i: (0, 0)),
        pl.BlockSpec((1, D), lambda i: (0, 0)),
    ],
    out_specs=[pl.BlockSpec((BN, D), lambda i: (i, 0))],
    out_shape=[jax.ShapeDtypeStruct((N, D), jnp.float32)],
)


def _tc3_body(h2_ref, wc_ref, bc_ref, head_ref, o_ref):
    hcT = lax.dot_general(wc_ref[...], h2_ref[...], (((1,), (1,)), ((), ())),
                          preferred_element_type=jnp.float32)
    hcT = hcT + bc_ref[:, 0:1]
    ca = jnp.cos(hcT)
    sa = jnp.sin(hcT)
    ca0, ca1, ca2, ca3 = (ca[i:i + 1, :] for i in range(4))
    sa0, sa1, sa2, sa3 = (sa[i:i + 1, :] for i in range(4))
    q = (head_ref[0] * (ca1 * ca3)
         + head_ref[1] * (ca0 * sa1 * sa2 * ca3)
         + head_ref[2] * (sa0 * sa2 * ca3)
         + head_ref[3] * (ca0 * ca2 * sa3)
         + head_ref[4] * (sa0 * sa1 * ca2 * sa3)
         + head_ref[5] * (ca1 * sa2)
         + head_ref[6] * (ca0 * sa1)
         + head_ref[7] * sa0)
    z = q * head_ref[8] + head_ref[9]
    o_ref[...] = 1.0 / (1.0 + jnp.exp(-z))


_tc3 = pl.pallas_call(
    _tc3_body,
    grid=(1,),
    in_specs=[
        pl.BlockSpec((N, D), lambda i: (0, 0)),
        pl.BlockSpec((8, D), lambda i: (0, 0)),
        pl.BlockSpec((8, D), lambda i: (0, 0)),
        pl.BlockSpec(memory_space=pltpu.SMEM),
    ],
    out_specs=[pl.BlockSpec((1, N), lambda i: (0, 0))],
    out_shape=[jax.ShapeDtypeStruct((1, N), jnp.float32)],
)


def kernel(x, edge_index, W1l, b1l, W1r, W2l, b2l, W2r, Wc, bc, theta, Wcl, bcl):
    src = edge_index[0]
    dst = edge_index[1]
    src2 = jnp.concatenate([src, jnp.zeros((EP - E,), jnp.int32)]).reshape(EP // CB, CB)
    dst2 = jnp.concatenate([dst, jnp.full((EP - E,), N, jnp.int32)]).reshape(EP // CB, CB)
    vals1 = jnp.concatenate(
        [x, jnp.ones((N, 1), jnp.float32), jnp.zeros((N, DP - D - 1), jnp.float32)],
        axis=1)

    a10, a11 = _make_sc_agg(DP)(vals1, src2, dst2)
    h1, inv = _tc1(a10, a11, x, W1l.T, W1r.T, b1l.reshape(1, D))
    a20, a21 = _make_sc_agg(D)(h1, src2, dst2)

    ct, st = jnp.cos(theta), jnp.sin(theta)
    head = jnp.stack([
        ct[0] * ct[1] * ct[2] * ct[3],
        -ct[0] * st[1] * ct[2] * ct[3],
        st[0] * st[1] * ct[2] * ct[3],
        -ct[0] * ct[1] * ct[2] * st[3],
        st[0] * ct[1] * ct[2] * st[3],
        ct[0] * ct[1] * st[2] * st[3],
        -ct[0] * st[1] * st[2] * st[3],
        st[0] * st[1] * st[2] * st[3],
        Wcl[0, 0], bcl[0],
        jnp.float32(0), jnp.float32(0), jnp.float32(0), jnp.float32(0),
        jnp.float32(0), jnp.float32(0),
    ])
    wc8 = jnp.zeros((8, D), jnp.float32).at[:4].set(Wc)
    bc8 = jnp.tile(jnp.pad(bc, (0, 4))[:, None], (1, D))

    (h2,) = _tc2(a20, a21, h1, inv, W2l.T, W2r.T, b2l.reshape(1, D))
    (oT,) = _tc3(h2, wc8, bc8, head)
    return oT.reshape(N, 1)
